# baseline (speedup 1.0000x reference)
import functools

import jax
import jax.numpy as jnp
from jax import lax
from jax.experimental import pallas as pl
from jax.experimental.pallas import tpu as pltpu
from jax.experimental.pallas import tpu_sc as plsc

_N, _H, _W, _C = 2, 256, 256, 96
_ROWS = _N * _H
_NPIX = _ROWS * _W
_NOPIX = 4 * _NPIX
_NWORKERS = 32
_ROWS_PER_W = _ROWS // _NWORKERS
_CHUNK_PIX = 64
_OCHUNK_PIX = 2 * _CHUNK_PIX
_CPR = _W // _CHUNK_PIX
_NCHUNK = _ROWS_PER_W * _CPR
_VPP = _C // 16
_UNROLL = 4


def _unpool_body(in_hbm, idx_hbm, out_hbm,
                 vin0, vin1, vidx0, vidx1, va0, vb0, va1, vb1,
                 sin0, sin1, sout0, sout1):
    vin = (vin0, vin1)
    vidx = (vidx0, vidx1)
    vout0 = (va0, va1)
    vout1 = (vb0, vb1)
    sin = (sin0, sin1)
    sout = (sout0, sout1)

    w = lax.axis_index("s") * 2 + lax.axis_index("c")

    def row_col(t):
        g = w * _ROWS_PER_W + t // _CPR
        n = g // _H
        i = g % _H
        col = (t % _CPR) * _CHUNK_PIX
        return n, i, col

    def start_in(t, b):
        n, i, col = row_col(t)
        pltpu.async_copy(in_hbm.at[n, i, pl.ds(col, _CHUNK_PIX), :], vin[b], sin[b])
        pltpu.async_copy(idx_hbm.at[n, i, pl.ds(col, _CHUNK_PIX), :], vidx[b], sin[b])

    def wait_in(b):
        pltpu.make_async_copy(in_hbm.at[0, 0, pl.ds(0, _CHUNK_PIX), :], vin[b],
                              sin[b]).wait()
        pltpu.make_async_copy(idx_hbm.at[0, 0, pl.ds(0, _CHUNK_PIX), :], vidx[b],
                              sin[b]).wait()

    def start_out(t, b):
        n, i, col = row_col(t)
        pltpu.async_copy(vout0[b], out_hbm.at[n, 2 * i, pl.ds(2 * col, _OCHUNK_PIX), :],
                         sout[b])
        pltpu.async_copy(vout1[b],
                         out_hbm.at[n, 2 * i + 1, pl.ds(2 * col, _OCHUNK_PIX), :],
                         sout[b])

    def wait_out(b):
        pltpu.make_async_copy(vout0[b], out_hbm.at[0, 0, pl.ds(0, _OCHUNK_PIX), :],
                              sout[b]).wait()
        pltpu.make_async_copy(vout1[b], out_hbm.at[0, 0, pl.ds(0, _OCHUNK_PIX), :],
                              sout[b]).wait()

    def compute(b):
        @plsc.parallel_loop(0, _CHUNK_PIX, unroll=_UNROLL)
        def _(p):
            for u in range(_VPP):
                cs = pl.ds(u * 16, 16)
                v = vin[b][p, cs]
                ix = vidx[b][p, cs]
                z = jnp.zeros((16,), jnp.float32)
                vout0[b][2 * p, cs] = jnp.where(ix == 0, v, z)
                vout0[b][2 * p + 1, cs] = jnp.where(ix == 1, v, z)
                vout1[b][2 * p, cs] = jnp.where(ix == 2, v, z)
                vout1[b][2 * p + 1, cs] = jnp.where(ix == 3, v, z)

    start_in(jnp.int32(0), 0)
    start_in(jnp.int32(1), 1)
    for b in range(2):
        t = jnp.int32(b)
        wait_in(b)
        compute(b)
        start_out(t, b)
        start_in(t + 2, b)

    def pair(i, carry):
        for b in range(2):
            t = 2 * i + b
            wait_in(b)
            wait_out(b)
            compute(b)
            start_out(t, b)
            tn = t + 2
            tn = jnp.where(tn < _NCHUNK, tn, 0)
            start_in(tn, b)
        return carry

    lax.fori_loop(1, _NCHUNK // 2, pair, 0)

    for b in range(2):
        wait_in(b)
        wait_out(b)


_mesh = plsc.VectorSubcoreMesh(core_axis_name="c", subcore_axis_name="s")

_unpool = functools.partial(
    pl.kernel,
    mesh=_mesh,
    out_type=jax.ShapeDtypeStruct((_N, 2 * _H, 2 * _W, _C), jnp.float32),
    compiler_params=pltpu.CompilerParams(use_tc_tiling_on_sc=True),
    scratch_types=[
        pltpu.VMEM((_CHUNK_PIX, _C), jnp.float32),
        pltpu.VMEM((_CHUNK_PIX, _C), jnp.float32),
        pltpu.VMEM((_CHUNK_PIX, _C), jnp.int32),
        pltpu.VMEM((_CHUNK_PIX, _C), jnp.int32),
        pltpu.VMEM((_OCHUNK_PIX, _C), jnp.float32),
        pltpu.VMEM((_OCHUNK_PIX, _C), jnp.float32),
        pltpu.VMEM((_OCHUNK_PIX, _C), jnp.float32),
        pltpu.VMEM((_OCHUNK_PIX, _C), jnp.float32),
        pltpu.SemaphoreType.DMA,
        pltpu.SemaphoreType.DMA,
        pltpu.SemaphoreType.DMA,
        pltpu.SemaphoreType.DMA,
    ],
)(_unpool_body)


@jax.jit
def kernel(inputs, indices):
    return _unpool(inputs, indices.astype(jnp.int32))


# speedup vs baseline: 3.5583x; 3.5583x over previous
import functools

import jax
import jax.numpy as jnp
from jax import lax
from jax.experimental import pallas as pl
from jax.experimental.pallas import tpu as pltpu
from jax.experimental.pallas import tpu_sc as plsc

_N, _H, _W, _C = 2, 256, 256, 96
_ROWS = _N * _H
_NWORKERS = 32
_ROWS_PER_W = _ROWS // _NWORKERS
_CK = 16
_CPS = _C // _CK
_NCHUNK = _ROWS_PER_W * _CPS
_NJ = _W // 16
_RING = 3
_UNROLL = 4


def _unpool_body(in_hbm, idx_hbm, out_hbm,
                 vin0, vin1, vin2, vidx0, vidx1, vidx2,
                 va0, va1, va2, vb0, vb1, vb2,
                 sin0, sin1, sin2, sout0, sout1, sout2):
    vin = (vin0, vin1, vin2)
    vidx = (vidx0, vidx1, vidx2)
    vout0 = (va0, va1, va2)
    vout1 = (vb0, vb1, vb2)
    sin = (sin0, sin1, sin2)
    sout = (sout0, sout1, sout2)

    w = lax.axis_index("s") * 2 + lax.axis_index("c")

    def pos(t):
        g = w * _ROWS_PER_W + t // _CPS
        n = g // _H
        i = g % _H
        c0 = (t % _CPS) * _CK
        return n, i, c0

    def start_in(t, b):
        n, i, c0 = pos(t)
        pltpu.async_copy(in_hbm.at[n, i, pl.ds(c0, _CK), :], vin[b], sin[b])
        pltpu.async_copy(idx_hbm.at[n, i, pl.ds(c0, _CK), :], vidx[b], sin[b])

    def wait_in(b):
        pltpu.make_async_copy(in_hbm.at[0, 0, pl.ds(0, _CK), :], vin[b],
                              sin[b]).wait()
        pltpu.make_async_copy(idx_hbm.at[0, 0, pl.ds(0, _CK), :], vidx[b],
                              sin[b]).wait()

    def start_out(t, b):
        n, i, c0 = pos(t)
        pltpu.async_copy(vout0[b], out_hbm.at[n, 2 * i, pl.ds(c0, _CK), :], sout[b])
        pltpu.async_copy(vout1[b], out_hbm.at[n, 2 * i + 1, pl.ds(c0, _CK), :],
                         sout[b])

    def wait_out(b):
        pltpu.make_async_copy(vout0[b], out_hbm.at[0, 0, pl.ds(0, _CK), :],
                              sout[b]).wait()
        pltpu.make_async_copy(vout1[b], out_hbm.at[0, 0, pl.ds(0, _CK), :],
                              sout[b]).wait()

    iota16 = lax.iota(jnp.int32, 16)
    z = jnp.zeros((16,), jnp.float32)

    def compute(b):
        @plsc.parallel_loop(0, _CK, unroll=_UNROLL)
        def _(c):
            row = jnp.full((16,), 0, jnp.int32) + c
            for q in range(_NJ):
                ws = pl.ds(q * 16, 16)
                v = vin[b][c, ws]
                ix = vidx[b][c, ws]
                col0 = 2 * q * 16 + 2 * iota16
                col1 = col0 + 1
                plsc.store_scatter(vout0[b], [row, col0], jnp.where(ix == 0, v, z))
                plsc.store_scatter(vout0[b], [row, col1], jnp.where(ix == 1, v, z))
                plsc.store_scatter(vout1[b], [row, col0], jnp.where(ix == 2, v, z))
                plsc.store_scatter(vout1[b], [row, col1], jnp.where(ix == 3, v, z))

    for b in range(_RING):
        start_in(jnp.int32(b), b)
    for b in range(_RING):
        t = jnp.int32(b)
        wait_in(b)
        compute(b)
        start_out(t, b)
        start_in(t + _RING, b)

    def group(i, carry):
        for b in range(_RING):
            t = _RING * i + b
            wait_in(b)
            wait_out(b)
            compute(b)
            start_out(t, b)
            tn = t + _RING
            tn = jnp.where(tn < _NCHUNK, tn, 0)
            start_in(tn, b)
        return carry

    lax.fori_loop(1, _NCHUNK // _RING, group, 0)

    for b in range(_RING):
        wait_in(b)
        wait_out(b)


_mesh = plsc.VectorSubcoreMesh(core_axis_name="c", subcore_axis_name="s")

_unpool = functools.partial(
    pl.kernel,
    mesh=_mesh,
    out_type=jax.ShapeDtypeStruct((_N, 2 * _H, _C, 2 * _W), jnp.float32),
    compiler_params=pltpu.CompilerParams(use_tc_tiling_on_sc=True,
                                         needs_layout_passes=False),
    scratch_types=(
        [pltpu.VMEM((_CK, _W), jnp.float32) for _ in range(_RING)]
        + [pltpu.VMEM((_CK, _W), jnp.int32) for _ in range(_RING)]
        + [pltpu.VMEM((_CK, 2 * _W), jnp.float32) for _ in range(2 * _RING)]
        + [pltpu.SemaphoreType.DMA for _ in range(2 * _RING)]
    ),
)(_unpool_body)


@jax.jit
def kernel(inputs, indices):
    in_t = jnp.transpose(inputs, (0, 1, 3, 2))
    idx_t = jnp.transpose(indices.astype(jnp.int32), (0, 1, 3, 2))
    out_t = _unpool(in_t, idx_t)
    return jnp.transpose(out_t, (0, 1, 3, 2))


# speedup vs baseline: 3.5773x; 1.0053x over previous
import functools

import jax
import jax.numpy as jnp
from jax import lax
from jax.experimental import pallas as pl
from jax.experimental.pallas import tpu as pltpu
from jax.experimental.pallas import tpu_sc as plsc

_N, _H, _W, _C = 2, 256, 256, 96
_ROWS = _N * _H
_NWORKERS = 32
_ROWS_PER_W = _ROWS // _NWORKERS
_CK = 32
_CPS = _C // _CK
_NCHUNK = _ROWS_PER_W * _CPS
_NJ = _W // 16
_UNROLL = 4


def _unpool_body(in_hbm, idx_hbm, out_hbm,
                 vin0, vin1, vidx0, vidx1, va0, vb0, va1, vb1,
                 sin0, sin1, sout0, sout1):
    vin = (vin0, vin1)
    vidx = (vidx0, vidx1)
    vout0 = (va0, va1)
    vout1 = (vb0, vb1)
    sin = (sin0, sin1)
    sout = (sout0, sout1)

    w = lax.axis_index("s") * 2 + lax.axis_index("c")

    def pos(t):
        g = w * _ROWS_PER_W + t // _CPS
        n = g // _H
        i = g % _H
        c0 = (t % _CPS) * _CK
        return n, i, c0

    def start_in(t, b):
        n, i, c0 = pos(t)
        pltpu.async_copy(in_hbm.at[n, i, pl.ds(c0, _CK), :], vin[b], sin[b])
        pltpu.async_copy(idx_hbm.at[n, i, pl.ds(c0, _CK), :], vidx[b], sin[b])

    def wait_in(b):
        pltpu.make_async_copy(in_hbm.at[0, 0, pl.ds(0, _CK), :], vin[b],
                              sin[b]).wait()
        pltpu.make_async_copy(idx_hbm.at[0, 0, pl.ds(0, _CK), :], vidx[b],
                              sin[b]).wait()

    def start_out(t, b):
        n, i, c0 = pos(t)
        pltpu.async_copy(vout0[b], out_hbm.at[n, 2 * i, pl.ds(c0, _CK), :], sout[b])
        pltpu.async_copy(vout1[b], out_hbm.at[n, 2 * i + 1, pl.ds(c0, _CK), :],
                         sout[b])

    def wait_out(b):
        pltpu.make_async_copy(vout0[b], out_hbm.at[0, 0, pl.ds(0, _CK), :],
                              sout[b]).wait()
        pltpu.make_async_copy(vout1[b], out_hbm.at[0, 0, pl.ds(0, _CK), :],
                              sout[b]).wait()

    iota16 = lax.iota(jnp.int32, 16)
    z = jnp.zeros((16,), jnp.float32)

    def compute(b):
        @plsc.parallel_loop(0, _CK, unroll=_UNROLL)
        def _(c):
            row = jnp.full((16,), 0, jnp.int32) + c
            for q in range(_NJ):
                ws = pl.ds(q * 16, 16)
                v = vin[b][c, ws]
                ix = vidx[b][c, ws]
                col0 = 2 * q * 16 + 2 * iota16
                col1 = col0 + 1
                plsc.store_scatter(vout0[b], [row, col0], jnp.where(ix == 0, v, z))
                plsc.store_scatter(vout0[b], [row, col1], jnp.where(ix == 1, v, z))
                plsc.store_scatter(vout1[b], [row, col0], jnp.where(ix == 2, v, z))
                plsc.store_scatter(vout1[b], [row, col1], jnp.where(ix == 3, v, z))

    start_in(jnp.int32(0), 0)
    start_in(jnp.int32(1), 1)
    for b in range(2):
        t = jnp.int32(b)
        wait_in(b)
        compute(b)
        start_out(t, b)
        start_in(t + 2, b)

    def pair(i, carry):
        for b in range(2):
            t = 2 * i + b
            wait_in(b)
            wait_out(b)
            compute(b)
            start_out(t, b)
            tn = t + 2
            tn = jnp.where(tn < _NCHUNK, tn, 0)
            start_in(tn, b)
        return carry

    lax.fori_loop(1, _NCHUNK // 2, pair, 0)

    for b in range(2):
        wait_in(b)
        wait_out(b)


_mesh = plsc.VectorSubcoreMesh(core_axis_name="c", subcore_axis_name="s")

_unpool = functools.partial(
    pl.kernel,
    mesh=_mesh,
    out_type=jax.ShapeDtypeStruct((_N, 2 * _H, _C, 2 * _W), jnp.float32),
    compiler_params=pltpu.CompilerParams(use_tc_tiling_on_sc=True,
                                         needs_layout_passes=False),
    scratch_types=[
        pltpu.VMEM((_CK, _W), jnp.float32),
        pltpu.VMEM((_CK, _W), jnp.float32),
        pltpu.VMEM((_CK, _W), jnp.int32),
        pltpu.VMEM((_CK, _W), jnp.int32),
        pltpu.VMEM((_CK, 2 * _W), jnp.float32),
        pltpu.VMEM((_CK, 2 * _W), jnp.float32),
        pltpu.VMEM((_CK, 2 * _W), jnp.float32),
        pltpu.VMEM((_CK, 2 * _W), jnp.float32),
        pltpu.SemaphoreType.DMA,
        pltpu.SemaphoreType.DMA,
        pltpu.SemaphoreType.DMA,
        pltpu.SemaphoreType.DMA,
    ],
)(_unpool_body)


@jax.jit
def kernel(inputs, indices):
    in_t = jnp.transpose(inputs, (0, 1, 3, 2))
    idx_t = jnp.transpose(indices.astype(jnp.int32), (0, 1, 3, 2))
    out_t = _unpool(in_t, idx_t)
    return jnp.transpose(out_t, (0, 1, 3, 2))
